# baseline (speedup 1.0000x reference)
import jax
import jax.numpy as jnp
from jax.experimental import pallas as pl


def _copy_body(x_ref, o_ref):
    o_ref[...] = x_ref[...]


def kernel(items):
    n = items.shape[0]
    chunks = n // 128
    rows = 2 * 3 * chunks * 2
    flat = (jnp.transpose(items, (1, 3, 0, 2))
            .reshape(2, 3, chunks, 128, 2)
            .transpose(0, 1, 2, 4, 3)
            .reshape(rows, 128))
    block_rows = 24576
    out = pl.pallas_call(
        _copy_body,
        grid=(rows // block_rows,),
        in_specs=[pl.BlockSpec((block_rows, 128), lambda i: (i, 0))],
        out_specs=pl.BlockSpec((block_rows, 128), lambda i: (i, 0)),
        out_shape=jax.ShapeDtypeStruct((rows, 128), jnp.float32),
    )(flat)
    return (out.reshape(2, 3, chunks, 2, 128)
            .transpose(0, 1, 2, 4, 3)
            .reshape(2, 3, n, 2)
            .transpose(2, 0, 3, 1))
